# baseline (speedup 1.0000x reference)
import functools

import jax
import jax.numpy as jnp
from jax import lax
from jax.experimental import pallas as pl
from jax.experimental.pallas import tpu as pltpu
from jax.experimental.pallas import tpu_sc as plsc

B, H, W = 8, 512, 512
N = B * H * W
NW = N // 2
SHIFT = 15
BIN_BASE = 27392
NB = 6144
NTILES = 32
WPT = NW // NTILES
UNROLL = 16
EPS = 1e-6
NEG_RATIO = 3.0


def _bce_body(pred_ref, gt_ref, mask_ref, packed_ref, part_ref):
    eps = jnp.float32(EPS)
    pred = pred_ref[...]
    gt = gt_ref[...]
    mask = mask_ref[...]
    p = jnp.clip(pred, eps, 1.0 - eps)
    loss = -jnp.log(jnp.where(gt > 0.5, p, 1.0 - p))
    positive = gt * mask
    negative = mask - positive
    bits = lax.bitcast_convert_type(negative * loss, jnp.int32)
    bin_ = jnp.clip(lax.shift_right_logical(bits, SHIFT) - BIN_BASE, 0, NB - 1)
    packed_ref[...] = jnp.concatenate(
        [bin_[:, :, 0:128] | (bin_[:, :, 128:256] << 16),
         bin_[:, :, 256:384] | (bin_[:, :, 384:512] << 16)], axis=1)
    pos_sum = jnp.sum(positive * loss)
    pos_cnt = jnp.sum(positive)
    neg_cnt = jnp.sum(mask) - pos_cnt
    lane = lax.broadcasted_iota(jnp.int32, (1, 1, 128), 2)
    part_ref[...] = (jnp.where(lane == 0, pos_sum, 0.0)
                     + jnp.where(lane == 1, pos_cnt, 0.0)
                     + jnp.where(lane == 2, neg_cnt, 0.0))


GRID = 16
RB = B * H // GRID


def _bce_call(pred, gt, mask):
    spec_img = pl.BlockSpec((1, RB, W), lambda i: (i, 0, 0))
    return pl.pallas_call(
        _bce_body,
        grid=(GRID,),
        in_specs=[spec_img, spec_img, spec_img],
        out_specs=[pl.BlockSpec((1, 2 * RB, 128), lambda i: (i, 0, 0)),
                   pl.BlockSpec((1, 1, 128), lambda i: (i, 0, 0))],
        out_shape=[
            jax.ShapeDtypeStruct((GRID, 2 * RB, 128), jnp.int32),
            jax.ShapeDtypeStruct((GRID, 1, 128), jnp.float32),
        ],
    )(pred.reshape(GRID, RB, W), gt.reshape(GRID, RB, W),
      mask.reshape(GRID, RB, W))


def _sc_hist_body(packed_hbm, counts_out, buf0, buf1, hist_c, sem0, sem1):
    nc = 2
    wid = lax.axis_index("s") * nc + lax.axis_index("c")
    half = WPT // 2
    base = wid * WPT
    cp0 = pltpu.async_copy(packed_hbm.at[pl.ds(base, half)], buf0, sem0)
    cp1 = pltpu.async_copy(packed_hbm.at[pl.ds(base + half, half)], buf1, sem1)

    zeros16 = jnp.zeros((16,), jnp.float32)

    def zero_body(i, carry):
        hist_c[pl.ds(i * 16, 16)] = zeros16
        return carry

    lax.fori_loop(0, NB // 16, zero_body, 0)

    ones = jnp.ones((16,), jnp.float32)

    for cp, buf in ((cp0, buf0), (cp1, buf1)):
        cp.wait()

        def step(j, carry, buf=buf):
            groups = []
            for u in range(UNROLL):
                w = buf[pl.ds((j * UNROLL + u) * 16, 16)]
                blo = w & 0xFFFF
                bhi = lax.shift_right_logical(w, 16)
                groups.append((blo, blo > 0, bhi, bhi > 0))
            for blo, mlo, bhi, mhi in groups:
                plsc.addupdate_scatter(hist_c, [blo], ones, mask=mlo)
                plsc.addupdate_scatter(hist_c, [bhi], ones, mask=mhi)
            return carry

        lax.fori_loop(0, half // (16 * UNROLL), step, 0)

    pltpu.sync_copy(hist_c, counts_out.at[wid])


def _sc_hist_call(packed_flat):
    mesh = plsc.VectorSubcoreMesh(core_axis_name="c", subcore_axis_name="s")
    fn = functools.partial(
        pl.kernel,
        out_type=jax.ShapeDtypeStruct((NTILES, NB), jnp.float32),
        mesh=mesh,
        scratch_types=[
            pltpu.VMEM((WPT // 2,), jnp.int32),
            pltpu.VMEM((WPT // 2,), jnp.int32),
            pltpu.VMEM((NB,), jnp.float32),
            pltpu.SemaphoreType.DMA,
            pltpu.SemaphoreType.DMA,
        ],
        compiler_params=pltpu.CompilerParams(needs_layout_passes=False),
    )(_sc_hist_body)
    return fn(packed_flat)


def _select_body(counts_ref, part_ref, out_ref):
    eps = jnp.float32(EPS)
    cnt = jnp.sum(counts_ref[...], axis=0)
    lane = lax.broadcasted_iota(jnp.int32, (GRID, 1, 128), 2)
    part = part_ref[...]
    pos_sum = jnp.sum(jnp.where(lane == 0, part, 0.0))
    pos_cnt = jnp.sum(jnp.where(lane == 1, part, 0.0))
    neg_cnt = jnp.sum(jnp.where(lane == 2, part, 0.0))
    k = jnp.minimum(neg_cnt, pos_cnt * NEG_RATIO)

    bin_id = lax.broadcasted_iota(jnp.int32, (8, NB), 1)
    gbin = bin_id + BIN_BASE
    scale = lax.bitcast_convert_type(
        jnp.left_shift((gbin >> 8) - 23, 23), jnp.float32)
    mid = scale * ((2.0 ** 23 + 2.0 ** 14)
                   + ((gbin & 255) << 15).astype(jnp.float32))
    sms = cnt * mid

    def search(i, lohi):
        lo, hi = lohi
        mid_ = (lo + hi) // 2
        c_ge = jnp.sum(jnp.where(bin_id >= mid_, cnt, 0.0))
        take_hi = c_ge >= k
        return (jnp.where(take_hi, mid_, lo), jnp.where(take_hi, hi, mid_))

    lo, hi = lax.fori_loop(0, 13, search, (jnp.int32(0), jnp.int32(NB)))
    c_above = jnp.sum(jnp.where(bin_id >= hi, cnt, 0.0))
    s_above = jnp.sum(jnp.where(bin_id >= hi, sms, 0.0))
    cb = jnp.sum(jnp.where(bin_id == lo, cnt, 0.0))
    sb = jnp.sum(jnp.where(bin_id == lo, sms, 0.0))
    partial = (k - c_above) * sb / jnp.maximum(cb, 1.0)
    neg_topk = s_above + partial
    bal = jnp.where(
        k > 0.0,
        (pos_sum + neg_topk) / (pos_cnt + k + eps),
        pos_sum / (pos_cnt + eps))
    olane = lax.broadcasted_iota(jnp.int32, (1, 128), 1)
    out_ref[...] = jnp.where(olane == 0, bal, 0.0)


def _select_call(counts, partials):
    return pl.pallas_call(
        _select_body,
        out_shape=jax.ShapeDtypeStruct((1, 128), jnp.float32),
    )(counts.reshape(NTILES // 8, 8, NB), partials)


def kernel(pred, gt, mask):
    packed, partials = _bce_call(pred, gt, mask)
    counts = _sc_hist_call(packed.reshape(NW))
    out = _select_call(counts, partials)
    return out[0, 0]


# speedup vs baseline: 1.0829x; 1.0829x over previous
import functools

import jax
import jax.numpy as jnp
from jax import lax
from jax.experimental import pallas as pl
from jax.experimental.pallas import tpu as pltpu
from jax.experimental.pallas import tpu_sc as plsc

B, H, W = 8, 512, 512
N = B * H * W
NW = N // 2
SHIFT = 15
BIN_BASE = 27392
NB = 6144
NTILES = 32
WPT = NW // NTILES
UNROLL = 16
EPS = 1e-6
NEG_RATIO = 3.0


def _bce_body(pred_ref, gt_ref, mask_ref, packed_ref, part_ref):
    eps = jnp.float32(EPS)
    pred = pred_ref[...]
    gt = gt_ref[...]
    mask = mask_ref[...]
    p = jnp.clip(pred, eps, 1.0 - eps)
    loss = -jnp.log(jnp.where(gt > 0.5, p, 1.0 - p))
    positive = gt * mask
    negative = mask - positive
    bits = lax.bitcast_convert_type(negative * loss, jnp.int32)
    bin_ = jnp.clip(lax.shift_right_logical(bits, SHIFT) - BIN_BASE, 0, NB - 1)
    packed_ref[...] = jnp.concatenate(
        [bin_[:, :, 0:128] | (bin_[:, :, 128:256] << 16),
         bin_[:, :, 256:384] | (bin_[:, :, 384:512] << 16)], axis=1)
    pos_sum = jnp.sum(positive * loss)
    pos_cnt = jnp.sum(positive)
    neg_cnt = jnp.sum(mask) - pos_cnt
    lane = lax.broadcasted_iota(jnp.int32, (1, 1, 128), 2)
    part_ref[...] = (jnp.where(lane == 0, pos_sum, 0.0)
                     + jnp.where(lane == 1, pos_cnt, 0.0)
                     + jnp.where(lane == 2, neg_cnt, 0.0))


GRID = 8
RB = B * H // GRID


def _bce_call(pred, gt, mask):
    spec_img = pl.BlockSpec((1, RB, W), lambda i: (i, 0, 0))
    return pl.pallas_call(
        _bce_body,
        grid=(GRID,),
        in_specs=[spec_img, spec_img, spec_img],
        out_specs=[pl.BlockSpec((1, 2 * RB, 128), lambda i: (i, 0, 0)),
                   pl.BlockSpec((1, 1, 128), lambda i: (i, 0, 0))],
        out_shape=[
            jax.ShapeDtypeStruct((GRID, 2 * RB, 128), jnp.int32),
            jax.ShapeDtypeStruct((GRID, 1, 128), jnp.float32),
        ],
    )(pred.reshape(GRID, RB, W), gt.reshape(GRID, RB, W),
      mask.reshape(GRID, RB, W))


def _sc_hist_body(packed_hbm, counts_out, buf0, buf1, hist_c, sem0, sem1):
    nc = 2
    wid = lax.axis_index("s") * nc + lax.axis_index("c")
    half = WPT // 2
    base = wid * WPT
    cp0 = pltpu.async_copy(packed_hbm.at[pl.ds(base, half)], buf0, sem0)
    cp1 = pltpu.async_copy(packed_hbm.at[pl.ds(base + half, half)], buf1, sem1)

    zeros16 = jnp.zeros((16,), jnp.float32)

    def zero_body(i, carry):
        hist_c[pl.ds(i * 16, 16)] = zeros16
        return carry

    lax.fori_loop(0, NB // 16, zero_body, 0)

    ones = jnp.ones((16,), jnp.float32)

    for cp, buf in ((cp0, buf0), (cp1, buf1)):
        cp.wait()

        def step(j, carry, buf=buf):
            groups = []
            for u in range(UNROLL):
                w = buf[pl.ds((j * UNROLL + u) * 16, 16)]
                blo = w & 0xFFFF
                bhi = lax.shift_right_logical(w, 16)
                groups.append((blo, blo > 0, bhi, bhi > 0))
            for blo, mlo, bhi, mhi in groups:
                plsc.addupdate_scatter(hist_c, [blo], ones, mask=mlo)
                plsc.addupdate_scatter(hist_c, [bhi], ones, mask=mhi)
            return carry

        lax.fori_loop(0, half // (16 * UNROLL), step, 0)

    pltpu.sync_copy(hist_c, counts_out.at[wid])


def _sc_hist_call(packed_flat):
    mesh = plsc.VectorSubcoreMesh(core_axis_name="c", subcore_axis_name="s")
    fn = functools.partial(
        pl.kernel,
        out_type=jax.ShapeDtypeStruct((NTILES, NB), jnp.float32),
        mesh=mesh,
        scratch_types=[
            pltpu.VMEM((WPT // 2,), jnp.int32),
            pltpu.VMEM((WPT // 2,), jnp.int32),
            pltpu.VMEM((NB,), jnp.float32),
            pltpu.SemaphoreType.DMA,
            pltpu.SemaphoreType.DMA,
        ],
        compiler_params=pltpu.CompilerParams(needs_layout_passes=False),
    )(_sc_hist_body)
    return fn(packed_flat)


def _select_body(counts_ref, part_ref, out_ref):
    eps = jnp.float32(EPS)
    cnt = jnp.sum(counts_ref[...], axis=0)
    lane = lax.broadcasted_iota(jnp.int32, (GRID, 1, 128), 2)
    part = part_ref[...]
    pos_sum = jnp.sum(jnp.where(lane == 0, part, 0.0))
    pos_cnt = jnp.sum(jnp.where(lane == 1, part, 0.0))
    neg_cnt = jnp.sum(jnp.where(lane == 2, part, 0.0))
    k = jnp.minimum(neg_cnt, pos_cnt * NEG_RATIO)

    bin_id = lax.broadcasted_iota(jnp.int32, (8, NB), 1)
    gbin = bin_id + BIN_BASE
    scale = lax.bitcast_convert_type(
        jnp.left_shift((gbin >> 8) - 23, 23), jnp.float32)
    mid = scale * ((2.0 ** 23 + 2.0 ** 14)
                   + ((gbin & 255) << 15).astype(jnp.float32))
    sms = cnt * mid

    def search(i, lohi):
        lo, hi = lohi
        mid_ = (lo + hi) // 2
        c_ge = jnp.sum(jnp.where(bin_id >= mid_, cnt, 0.0))
        take_hi = c_ge >= k
        return (jnp.where(take_hi, mid_, lo), jnp.where(take_hi, hi, mid_))

    lo, hi = lax.fori_loop(0, 13, search, (jnp.int32(0), jnp.int32(NB)))
    c_above = jnp.sum(jnp.where(bin_id >= hi, cnt, 0.0))
    s_above = jnp.sum(jnp.where(bin_id >= hi, sms, 0.0))
    cb = jnp.sum(jnp.where(bin_id == lo, cnt, 0.0))
    sb = jnp.sum(jnp.where(bin_id == lo, sms, 0.0))
    partial = (k - c_above) * sb / jnp.maximum(cb, 1.0)
    neg_topk = s_above + partial
    bal = jnp.where(
        k > 0.0,
        (pos_sum + neg_topk) / (pos_cnt + k + eps),
        pos_sum / (pos_cnt + eps))
    olane = lax.broadcasted_iota(jnp.int32, (1, 128), 1)
    out_ref[...] = jnp.where(olane == 0, bal, 0.0)


def _select_call(counts, partials):
    return pl.pallas_call(
        _select_body,
        out_shape=jax.ShapeDtypeStruct((1, 128), jnp.float32),
    )(counts.reshape(NTILES // 8, 8, NB), partials)


def kernel(pred, gt, mask):
    packed, partials = _bce_call(pred, gt, mask)
    counts = _sc_hist_call(packed.reshape(NW))
    out = _select_call(counts, partials)
    return out[0, 0]


# speedup vs baseline: 1.0860x; 1.0028x over previous
import functools

import jax
import jax.numpy as jnp
from jax import lax
from jax.experimental import pallas as pl
from jax.experimental.pallas import tpu as pltpu
from jax.experimental.pallas import tpu_sc as plsc

B, H, W = 8, 512, 512
N = B * H * W
NW = N // 2
SHIFT = 15
BIN_BASE = 27392
NB = 6144
NTILES = 32
WPT = NW // NTILES
UNROLL = 32
EPS = 1e-6
NEG_RATIO = 3.0


def _bce_body(pred_ref, gt_ref, mask_ref, packed_ref, part_ref):
    eps = jnp.float32(EPS)
    pred = pred_ref[...]
    gt = gt_ref[...]
    mask = mask_ref[...]
    p = jnp.clip(pred, eps, 1.0 - eps)
    loss = -jnp.log(jnp.where(gt > 0.5, p, 1.0 - p))
    positive = gt * mask
    negative = mask - positive
    bits = lax.bitcast_convert_type(negative * loss, jnp.int32)
    bin_ = jnp.maximum(lax.shift_right_logical(bits, SHIFT) - BIN_BASE, 0)
    packed_ref[...] = jnp.concatenate(
        [bin_[:, :, 0:128] | (bin_[:, :, 128:256] << 16),
         bin_[:, :, 256:384] | (bin_[:, :, 384:512] << 16)], axis=1)
    pos_sum = jnp.sum(positive * loss)
    pos_cnt = jnp.sum(positive)
    neg_cnt = jnp.sum(mask) - pos_cnt
    lane = lax.broadcasted_iota(jnp.int32, (1, 1, 128), 2)
    part_ref[...] = (jnp.where(lane == 0, pos_sum, 0.0)
                     + jnp.where(lane == 1, pos_cnt, 0.0)
                     + jnp.where(lane == 2, neg_cnt, 0.0))


GRID = 8
RB = B * H // GRID


def _bce_call(pred, gt, mask):
    spec_img = pl.BlockSpec((1, RB, W), lambda i: (i, 0, 0))
    return pl.pallas_call(
        _bce_body,
        grid=(GRID,),
        in_specs=[spec_img, spec_img, spec_img],
        out_specs=[pl.BlockSpec((1, 2 * RB, 128), lambda i: (i, 0, 0)),
                   pl.BlockSpec((1, 1, 128), lambda i: (i, 0, 0))],
        out_shape=[
            jax.ShapeDtypeStruct((GRID, 2 * RB, 128), jnp.int32),
            jax.ShapeDtypeStruct((GRID, 1, 128), jnp.float32),
        ],
    )(pred.reshape(GRID, RB, W), gt.reshape(GRID, RB, W),
      mask.reshape(GRID, RB, W))


def _sc_hist_body(packed_hbm, counts_out, buf0, buf1, hist_c, sem0, sem1):
    nc = 2
    wid = lax.axis_index("s") * nc + lax.axis_index("c")
    half = WPT // 2
    base = wid * WPT
    cp0 = pltpu.async_copy(packed_hbm.at[pl.ds(base, half)], buf0, sem0)
    cp1 = pltpu.async_copy(packed_hbm.at[pl.ds(base + half, half)], buf1, sem1)

    zeros16 = jnp.zeros((16,), jnp.float32)

    def zero_body(i, carry):
        for u in range(8):
            hist_c[pl.ds((i * 8 + u) * 16, 16)] = zeros16
        return carry

    lax.fori_loop(0, NB // 128, zero_body, 0)

    ones = jnp.ones((16,), jnp.float32)

    for cp, buf in ((cp0, buf0), (cp1, buf1)):
        cp.wait()

        def step(j, carry, buf=buf):
            groups = []
            for u in range(UNROLL):
                w = buf[pl.ds((j * UNROLL + u) * 16, 16)]
                blo = w & 0xFFFF
                bhi = lax.shift_right_logical(w, 16)
                groups.append((blo, blo > 0, bhi, bhi > 0))
            for blo, mlo, bhi, mhi in groups:
                plsc.addupdate_scatter(hist_c, [blo], ones, mask=mlo)
                plsc.addupdate_scatter(hist_c, [bhi], ones, mask=mhi)
            return carry

        lax.fori_loop(0, half // (16 * UNROLL), step, 0)

    pltpu.sync_copy(hist_c, counts_out.at[wid])


def _sc_hist_call(packed_flat):
    mesh = plsc.VectorSubcoreMesh(core_axis_name="c", subcore_axis_name="s")
    fn = functools.partial(
        pl.kernel,
        out_type=jax.ShapeDtypeStruct((NTILES, NB), jnp.float32),
        mesh=mesh,
        scratch_types=[
            pltpu.VMEM((WPT // 2,), jnp.int32),
            pltpu.VMEM((WPT // 2,), jnp.int32),
            pltpu.VMEM((NB,), jnp.float32),
            pltpu.SemaphoreType.DMA,
            pltpu.SemaphoreType.DMA,
        ],
        compiler_params=pltpu.CompilerParams(needs_layout_passes=False),
    )(_sc_hist_body)
    return fn(packed_flat)


def _select_body(counts_ref, part_ref, out_ref):
    eps = jnp.float32(EPS)
    cnt = jnp.sum(counts_ref[...], axis=0)
    lane = lax.broadcasted_iota(jnp.int32, (GRID, 1, 128), 2)
    part = part_ref[...]
    pos_sum = jnp.sum(jnp.where(lane == 0, part, 0.0))
    pos_cnt = jnp.sum(jnp.where(lane == 1, part, 0.0))
    neg_cnt = jnp.sum(jnp.where(lane == 2, part, 0.0))
    k = jnp.minimum(neg_cnt, pos_cnt * NEG_RATIO)

    bin_id = lax.broadcasted_iota(jnp.int32, (8, NB), 1)
    gbin = bin_id + BIN_BASE
    scale = lax.bitcast_convert_type(
        jnp.left_shift((gbin >> 8) - 23, 23), jnp.float32)
    mid = scale * ((2.0 ** 23 + 2.0 ** 14)
                   + ((gbin & 255) << 15).astype(jnp.float32))
    sms = cnt * mid

    def search(i, lohi):
        lo, hi = lohi
        mid_ = (lo + hi) // 2
        c_ge = jnp.sum(jnp.where(bin_id >= mid_, cnt, 0.0))
        take_hi = c_ge >= k
        return (jnp.where(take_hi, mid_, lo), jnp.where(take_hi, hi, mid_))

    lo, hi = lax.fori_loop(0, 13, search, (jnp.int32(0), jnp.int32(NB)))
    c_above = jnp.sum(jnp.where(bin_id >= hi, cnt, 0.0))
    s_above = jnp.sum(jnp.where(bin_id >= hi, sms, 0.0))
    cb = jnp.sum(jnp.where(bin_id == lo, cnt, 0.0))
    sb = jnp.sum(jnp.where(bin_id == lo, sms, 0.0))
    partial = (k - c_above) * sb / jnp.maximum(cb, 1.0)
    neg_topk = s_above + partial
    bal = jnp.where(
        k > 0.0,
        (pos_sum + neg_topk) / (pos_cnt + k + eps),
        pos_sum / (pos_cnt + eps))
    olane = lax.broadcasted_iota(jnp.int32, (1, 128), 1)
    out_ref[...] = jnp.where(olane == 0, bal, 0.0)


def _select_call(counts, partials):
    return pl.pallas_call(
        _select_body,
        out_shape=jax.ShapeDtypeStruct((1, 128), jnp.float32),
    )(counts.reshape(NTILES // 8, 8, NB), partials)


def kernel(pred, gt, mask):
    packed, partials = _bce_call(pred, gt, mask)
    counts = _sc_hist_call(packed.reshape(NW))
    out = _select_call(counts, partials)
    return out[0, 0]


# speedup vs baseline: 1.1110x; 1.0231x over previous
import functools

import jax
import jax.numpy as jnp
from jax import lax
from jax.experimental import pallas as pl
from jax.experimental.pallas import tpu as pltpu
from jax.experimental.pallas import tpu_sc as plsc

B, H, W = 8, 512, 512
N = B * H * W
NW = N // 2
SHIFT = 15
BIN_BASE = 27392
NB = 6144
NTILES = 32
WPT = NW // NTILES
UNROLL = 32
EPS = 1e-6
NEG_RATIO = 3.0


def _bce_body(pred_ref, gt_ref, mask_ref, packed_ref, part_ref):
    eps = jnp.float32(EPS)
    pred = pred_ref[...]
    gt = gt_ref[...]
    mask = mask_ref[...]
    p = jnp.clip(pred, eps, 1.0 - eps)
    loss = -jnp.log(jnp.where(gt > 0.5, p, 1.0 - p))
    positive = gt * mask
    negative = mask - positive
    bits = lax.bitcast_convert_type(negative * loss, jnp.int32)
    bin_ = jnp.maximum(lax.shift_right_logical(bits, SHIFT) - BIN_BASE, 0)
    packed_ref[...] = jnp.concatenate(
        [bin_[:, :, 0:128] | (bin_[:, :, 128:256] << 16),
         bin_[:, :, 256:384] | (bin_[:, :, 384:512] << 16)], axis=1)
    pos_sum = jnp.sum(positive * loss)
    pos_cnt = jnp.sum(positive)
    neg_cnt = jnp.sum(mask) - pos_cnt
    lane = lax.broadcasted_iota(jnp.int32, (1, 1, 128), 2)
    part_ref[...] = (jnp.where(lane == 0, pos_sum, 0.0)
                     + jnp.where(lane == 1, pos_cnt, 0.0)
                     + jnp.where(lane == 2, neg_cnt, 0.0))


GRID = 4
RB = B * H // GRID


def _bce_call(pred, gt, mask):
    spec_img = pl.BlockSpec((1, RB, W), lambda i: (i, 0, 0))
    return pl.pallas_call(
        _bce_body,
        grid=(GRID,),
        in_specs=[spec_img, spec_img, spec_img],
        out_specs=[pl.BlockSpec((1, 2 * RB, 128), lambda i: (i, 0, 0)),
                   pl.BlockSpec((1, 1, 128), lambda i: (i, 0, 0))],
        out_shape=[
            jax.ShapeDtypeStruct((GRID, 2 * RB, 128), jnp.int32),
            jax.ShapeDtypeStruct((GRID, 1, 128), jnp.float32),
        ],
    )(pred.reshape(GRID, RB, W), gt.reshape(GRID, RB, W),
      mask.reshape(GRID, RB, W))


def _sc_hist_body(packed_hbm, counts_out, buf0, buf1, hist_c, sem0, sem1):
    nc = 2
    wid = lax.axis_index("s") * nc + lax.axis_index("c")
    half = WPT // 2
    base = wid * WPT
    cp0 = pltpu.async_copy(packed_hbm.at[pl.ds(base, half)], buf0, sem0)
    cp1 = pltpu.async_copy(packed_hbm.at[pl.ds(base + half, half)], buf1, sem1)

    zeros16 = jnp.zeros((16,), jnp.float32)

    def zero_body(i, carry):
        for u in range(8):
            hist_c[pl.ds((i * 8 + u) * 16, 16)] = zeros16
        return carry

    lax.fori_loop(0, NB // 128, zero_body, 0)

    ones = jnp.ones((16,), jnp.float32)

    for cp, buf in ((cp0, buf0), (cp1, buf1)):
        cp.wait()

        def step(j, carry, buf=buf):
            groups = []
            for u in range(UNROLL):
                w = buf[pl.ds((j * UNROLL + u) * 16, 16)]
                blo = w & 0xFFFF
                bhi = lax.shift_right_logical(w, 16)
                groups.append((blo, blo > 0, bhi, bhi > 0))
            for blo, mlo, bhi, mhi in groups:
                plsc.addupdate_scatter(hist_c, [blo], ones, mask=mlo)
                plsc.addupdate_scatter(hist_c, [bhi], ones, mask=mhi)
            return carry

        lax.fori_loop(0, half // (16 * UNROLL), step, 0)

    pltpu.sync_copy(hist_c, counts_out.at[wid])


def _sc_hist_call(packed_flat):
    mesh = plsc.VectorSubcoreMesh(core_axis_name="c", subcore_axis_name="s")
    fn = functools.partial(
        pl.kernel,
        out_type=jax.ShapeDtypeStruct((NTILES, NB), jnp.float32),
        mesh=mesh,
        scratch_types=[
            pltpu.VMEM((WPT // 2,), jnp.int32),
            pltpu.VMEM((WPT // 2,), jnp.int32),
            pltpu.VMEM((NB,), jnp.float32),
            pltpu.SemaphoreType.DMA,
            pltpu.SemaphoreType.DMA,
        ],
        compiler_params=pltpu.CompilerParams(needs_layout_passes=False),
    )(_sc_hist_body)
    return fn(packed_flat)


def _select_body(counts_ref, part_ref, out_ref):
    eps = jnp.float32(EPS)
    cnt = jnp.sum(counts_ref[...], axis=0)
    lane = lax.broadcasted_iota(jnp.int32, (GRID, 1, 128), 2)
    part = part_ref[...]
    pos_sum = jnp.sum(jnp.where(lane == 0, part, 0.0))
    pos_cnt = jnp.sum(jnp.where(lane == 1, part, 0.0))
    neg_cnt = jnp.sum(jnp.where(lane == 2, part, 0.0))
    k = jnp.minimum(neg_cnt, pos_cnt * NEG_RATIO)

    bin_id = lax.broadcasted_iota(jnp.int32, (8, NB), 1)
    gbin = bin_id + BIN_BASE
    scale = lax.bitcast_convert_type(
        jnp.left_shift((gbin >> 8) - 23, 23), jnp.float32)
    mid = scale * ((2.0 ** 23 + 2.0 ** 14)
                   + ((gbin & 255) << 15).astype(jnp.float32))
    sms = cnt * mid

    def search(i, lohi):
        lo, hi = lohi
        mid_ = (lo + hi) // 2
        c_ge = jnp.sum(jnp.where(bin_id >= mid_, cnt, 0.0))
        take_hi = c_ge >= k
        return (jnp.where(take_hi, mid_, lo), jnp.where(take_hi, hi, mid_))

    lo, hi = lax.fori_loop(0, 13, search, (jnp.int32(0), jnp.int32(NB)))
    c_above = jnp.sum(jnp.where(bin_id >= hi, cnt, 0.0))
    s_above = jnp.sum(jnp.where(bin_id >= hi, sms, 0.0))
    cb = jnp.sum(jnp.where(bin_id == lo, cnt, 0.0))
    sb = jnp.sum(jnp.where(bin_id == lo, sms, 0.0))
    partial = (k - c_above) * sb / jnp.maximum(cb, 1.0)
    neg_topk = s_above + partial
    bal = jnp.where(
        k > 0.0,
        (pos_sum + neg_topk) / (pos_cnt + k + eps),
        pos_sum / (pos_cnt + eps))
    olane = lax.broadcasted_iota(jnp.int32, (1, 128), 1)
    out_ref[...] = jnp.where(olane == 0, bal, 0.0)


def _select_call(counts, partials):
    return pl.pallas_call(
        _select_body,
        out_shape=jax.ShapeDtypeStruct((1, 128), jnp.float32),
    )(counts.reshape(NTILES // 8, 8, NB), partials)


def kernel(pred, gt, mask):
    packed, partials = _bce_call(pred, gt, mask)
    counts = _sc_hist_call(packed.reshape(NW))
    out = _select_call(counts, partials)
    return out[0, 0]
